# Initial kernel scaffold; baseline (speedup 1.0000x reference)
import jax
import jax.numpy as jnp
from jax.experimental import pallas as pl


def kernel(atom_feats, bond_feats, global_feats, atom_segment_ids, bond_segment_ids):
    raise NotImplementedError("write your pallas kernel here")


# speedup vs baseline: 4.3220x; 4.3220x over previous
import functools

import jax
import jax.numpy as jnp
from jax import lax
from jax.experimental import pallas as pl
from jax.experimental.pallas import tpu as pltpu
from jax.experimental.pallas import tpu_sc as plsc

N = 100000
D = 128
G = 1024
CHUNK = 128
NFULL = N // CHUNK
TAIL = N - NFULL * CHUNK
NSUB = 16
ITERS = -(-NFULL // NSUB)
GROWS = G // NSUB


def _segment_sum_body(sid, feats, ids, out, acc, rows_v, idx_v, rows_t, idx_t):
    def zero_row(r, _):
        for j in range(D // 16):
            rows_v[r, pl.ds(j * 16, 16)] = jnp.zeros((16,), jnp.float32)
        return _

    lax.fori_loop(0, GROWS, zero_row, None)
    pltpu.sync_copy(rows_v.at[pl.ds(0, GROWS)], acc.at[pl.ds(sid * GROWS, GROWS)])
    plsc.subcore_barrier()

    def body(i, _):
        c = sid + i * NSUB

        @pl.when(c < NFULL)
        def _():
            off = c * CHUNK
            pltpu.sync_copy(feats.at[pl.ds(off, CHUNK)], rows_v)
            pltpu.sync_copy(ids.at[pl.ds(off, CHUNK)], idx_v)
            pltpu.sync_copy(rows_v, acc.at[idx_v], add=True)

        return _

    lax.fori_loop(0, ITERS, body, None)

    @pl.when(sid == NSUB - 1)
    def _():
        off = NFULL * CHUNK
        pltpu.sync_copy(feats.at[pl.ds(off, TAIL)], rows_t)
        pltpu.sync_copy(ids.at[pl.ds(off, TAIL)], idx_t)
        pltpu.sync_copy(rows_t, acc.at[idx_t], add=True)

    plsc.subcore_barrier()
    pltpu.sync_copy(acc.at[pl.ds(sid * GROWS, GROWS)], out.at[pl.ds(sid * GROWS, GROWS)])


@functools.partial(
    pl.kernel,
    out_type=(
        jax.ShapeDtypeStruct((G, D), jnp.float32),
        jax.ShapeDtypeStruct((G, D), jnp.float32),
    ),
    mesh=plsc.VectorSubcoreMesh(
        core_axis_name="c", subcore_axis_name="s", num_cores=2, num_subcores=NSUB
    ),
    scratch_types=(
        pltpu.VMEM_SHARED((G, D), jnp.float32),
        pltpu.VMEM((CHUNK, D), jnp.float32),
        pltpu.VMEM((CHUNK,), jnp.int32),
        pltpu.VMEM((TAIL, D), jnp.float32),
        pltpu.VMEM((TAIL,), jnp.int32),
    ),
)
def _pooled(atom_hbm, aids_hbm, bond_hbm, bids_hbm, atom_out, bond_out,
            acc, rows_v, idx_v, rows_t, idx_t):
    cid = lax.axis_index("c")
    sid = lax.axis_index("s")

    @pl.when(cid == 0)
    def _():
        _segment_sum_body(sid, atom_hbm, aids_hbm, atom_out,
                          acc, rows_v, idx_v, rows_t, idx_t)

    @pl.when(cid == 1)
    def _():
        _segment_sum_body(sid, bond_hbm, bids_hbm, bond_out,
                          acc, rows_v, idx_v, rows_t, idx_t)


def kernel(atom_feats, bond_feats, global_feats, atom_segment_ids, bond_segment_ids):
    atom_pool, bond_pool = _pooled(
        atom_feats, atom_segment_ids, bond_feats, bond_segment_ids
    )
    return jnp.concatenate([atom_pool, bond_pool, global_feats], axis=-1)


# speedup vs baseline: 6.7718x; 1.5668x over previous
import functools

import jax
import jax.numpy as jnp
from jax import lax
from jax.experimental import pallas as pl
from jax.experimental.pallas import tpu as pltpu
from jax.experimental.pallas import tpu_sc as plsc

N = 100000
D = 128
G = 1024
CHUNK = 128
NFULL = N // CHUNK
TAIL = N - NFULL * CHUNK
NSUB = 16
ITERS = -(-NFULL // NSUB)
GROWS = G // NSUB


def _segment_sum_body(sid, feats, ids, out, acc,
                      rows_a, rows_b, idx_a, idx_b, sem_a, sem_b,
                      rows_t, idx_t):
    def zero_row(r, _):
        for j in range(D // 16):
            rows_a[r, pl.ds(j * 16, 16)] = jnp.zeros((16,), jnp.float32)
        return _

    lax.fori_loop(0, GROWS, zero_row, None)
    pltpu.sync_copy(rows_a.at[pl.ds(0, GROWS)], acc.at[pl.ds(sid * GROWS, GROWS)])
    plsc.subcore_barrier()

    slot_a = (rows_a, idx_a, sem_a)
    slot_b = (rows_b, idx_b, sem_b)

    def issue(slot, c):
        rows, idx, sem = slot
        off = c * CHUNK
        pltpu.async_copy(feats.at[pl.ds(off, CHUNK)], rows, sem)
        pltpu.async_copy(ids.at[pl.ds(off, CHUNK)], idx, sem)

    def drain(slot, c):
        rows, idx, sem = slot
        off = c * CHUNK
        pltpu.make_async_copy(feats.at[pl.ds(off, CHUNK)], rows, sem).wait()
        pltpu.make_async_copy(ids.at[pl.ds(off, CHUNK)], idx, sem).wait()

    issue(slot_a, sid)

    def step(i, cur, nxt):
        c = sid + i * NSUB
        cn = c + NSUB

        @pl.when(c < NFULL)
        def _():
            drain(cur, c)

        @pl.when(cn < NFULL)
        def _():
            issue(nxt, cn)

        @pl.when(c < NFULL)
        def _():
            pltpu.sync_copy(cur[0], acc.at[cur[1]], add=True)

    def body2(i2, _):
        step(i2 * 2, slot_a, slot_b)
        step(i2 * 2 + 1, slot_b, slot_a)
        return _

    lax.fori_loop(0, (ITERS + 1) // 2, body2, None)

    @pl.when(sid == NSUB - 1)
    def _():
        off = NFULL * CHUNK
        pltpu.sync_copy(feats.at[pl.ds(off, TAIL)], rows_t)
        pltpu.sync_copy(ids.at[pl.ds(off, TAIL)], idx_t)
        pltpu.sync_copy(rows_t, acc.at[idx_t], add=True)

    plsc.subcore_barrier()
    pltpu.sync_copy(acc.at[pl.ds(sid * GROWS, GROWS)], out.at[pl.ds(sid * GROWS, GROWS)])


@functools.partial(
    pl.kernel,
    out_type=(
        jax.ShapeDtypeStruct((G, D), jnp.float32),
        jax.ShapeDtypeStruct((G, D), jnp.float32),
    ),
    mesh=plsc.VectorSubcoreMesh(
        core_axis_name="c", subcore_axis_name="s", num_cores=2, num_subcores=NSUB
    ),
    scratch_types=(
        pltpu.VMEM_SHARED((G, D), jnp.float32),
        pltpu.VMEM((CHUNK, D), jnp.float32),
        pltpu.VMEM((CHUNK, D), jnp.float32),
        pltpu.VMEM((CHUNK,), jnp.int32),
        pltpu.VMEM((CHUNK,), jnp.int32),
        pltpu.SemaphoreType.DMA,
        pltpu.SemaphoreType.DMA,
        pltpu.VMEM((TAIL, D), jnp.float32),
        pltpu.VMEM((TAIL,), jnp.int32),
    ),
)
def _pooled(atom_hbm, aids_hbm, bond_hbm, bids_hbm, atom_out, bond_out,
            acc, rows_a, rows_b, idx_a, idx_b, sem_a, sem_b, rows_t, idx_t):
    cid = lax.axis_index("c")
    sid = lax.axis_index("s")

    @pl.when(cid == 0)
    def _():
        _segment_sum_body(sid, atom_hbm, aids_hbm, atom_out, acc,
                          rows_a, rows_b, idx_a, idx_b, sem_a, sem_b,
                          rows_t, idx_t)

    @pl.when(cid == 1)
    def _():
        _segment_sum_body(sid, bond_hbm, bids_hbm, bond_out, acc,
                          rows_a, rows_b, idx_a, idx_b, sem_a, sem_b,
                          rows_t, idx_t)


def kernel(atom_feats, bond_feats, global_feats, atom_segment_ids, bond_segment_ids):
    atom_pool, bond_pool = _pooled(
        atom_feats, atom_segment_ids, bond_feats, bond_segment_ids
    )
    return jnp.concatenate([atom_pool, bond_pool, global_feats], axis=-1)
